# baseline (speedup 1.0000x reference)
import jax
import jax.numpy as jnp
from jax import lax
from jax.experimental import pallas as pl
from jax.experimental.pallas import tpu as pltpu
from jax.experimental.pallas import tpu_sc as plsc

VOCAB = 100000
EMBED = 128
MAXLEN = 512
BATCH = 1024

NC = 2
NS = 16
LANES = 16
NW = NC * NS

N_ROWS = BATCH * MAXLEN
ROWS_PER_W = N_ROWS // NW
CHUNK = 64
CHUNKS_PER_W = ROWS_PER_W // CHUNK
POS_PERIOD = MAXLEN // CHUNK
NBUF = 4
AHEAD = 2
ITERS = CHUNKS_PER_W // NBUF


def _body(x_hbm, tok_hbm, pos_hbm, out_hbm,
          idx_v, r0, r1, r2, r3, pos_v,
          g0, g1, g2, g3, s0, s1, s2, s3):
    rows = [r0, r1, r2, r3]
    gsem = [g0, g1, g2, g3]
    ssem = [s0, s1, s2, s3]

    wid = lax.axis_index("s") * NC + lax.axis_index("c")
    cbase = wid * CHUNKS_PER_W
    rbase = wid * ROWS_PER_W

    pltpu.sync_copy(x_hbm.at[pl.ds(cbase, CHUNKS_PER_W)], idx_v)
    pltpu.sync_copy(pos_hbm, pos_v)

    def start_gather(c, b):
        pltpu.async_copy(tok_hbm.at[idx_v.at[c]], rows[b], gsem[b])

    def wait_gather(c, b):
        pltpu.make_async_copy(tok_hbm.at[idx_v.at[c]], rows[b], gsem[b]).wait()

    def start_store(c, b):
        pltpu.async_copy(
            rows[b], out_hbm.at[pl.ds(rbase + c * CHUNK, CHUNK)], ssem[b])

    def wait_store(c, b):
        pltpu.make_async_copy(
            rows[b], out_hbm.at[pl.ds(rbase + c * CHUNK, CHUNK)],
            ssem[b]).wait()

    def add_pos(c, b):
        p0 = lax.rem(c, POS_PERIOD) * CHUNK

        @plsc.parallel_loop(0, CHUNK, unroll=8)
        def _add(r):
            for j in range(EMBED // LANES):
                sl = pl.ds(j * LANES, LANES)
                plsc.addupdate(rows[b].at[r, sl], pos_v[p0 + r, sl])

    for b in range(AHEAD):
        start_gather(b, b)

    for b in range(NBUF):
        c = b
        wait_gather(c, b)
        add_pos(c, b)
        start_store(c, b)
        b2 = (b + AHEAD) % NBUF
        if c + AHEAD >= NBUF:
            wait_store(c + AHEAD - NBUF, b2)
        start_gather(c + AHEAD, b2)

    def iter_body(i, carry):
        c0 = i * NBUF
        for b in range(NBUF):
            c = c0 + b
            wait_gather(c, b)
            add_pos(c, b)
            start_store(c, b)
            b2 = (b + AHEAD) % NBUF

            @pl.when(c + AHEAD < CHUNKS_PER_W)
            def _():
                wait_store(c + AHEAD - NBUF, b2)
                start_gather(c + AHEAD, b2)
        return carry

    lax.fori_loop(1, ITERS, iter_body, 0)

    for b in range(NBUF):
        wait_store(CHUNKS_PER_W - NBUF + b, b)


@jax.jit
def _embed(idx, token_table, pos_table):
    mesh = plsc.VectorSubcoreMesh(core_axis_name="c", subcore_axis_name="s")
    return pl.kernel(
        _body,
        out_type=jax.ShapeDtypeStruct((N_ROWS, EMBED), jnp.float32),
        mesh=mesh,
        scratch_types=[
            pltpu.VMEM((CHUNKS_PER_W, CHUNK), jnp.int32),
        ] + [pltpu.VMEM((CHUNK, EMBED), jnp.float32) for _ in range(NBUF)] + [
            pltpu.VMEM((MAXLEN, EMBED), jnp.float32),
        ] + [pltpu.SemaphoreType.DMA for _ in range(2 * NBUF)],
    )(idx, token_table, pos_table)


def kernel(x, token_table, pos_table):
    b, m = x.shape
    idx = x.reshape(N_ROWS // CHUNK, CHUNK).astype(jnp.int32)
    out = _embed(idx, token_table, pos_table)
    return out.reshape(b, m, EMBED)


# speedup vs baseline: 1.1879x; 1.1879x over previous
import jax
import jax.numpy as jnp
from jax import lax
from jax.experimental import pallas as pl
from jax.experimental.pallas import tpu as pltpu
from jax.experimental.pallas import tpu_sc as plsc

VOCAB = 100000
EMBED = 128
MAXLEN = 512
BATCH = 1024

NC = 2
NS = 16
LANES = 16
NW = NC * NS

N_ROWS = BATCH * MAXLEN
POS_PER_W = MAXLEN // NW
ROWS_PER_W = N_ROWS // NW
BB = 8
CHUNK = BB * POS_PER_W
CHUNKS_PER_W = BATCH // BB
NBUF = 4
AHEAD = 2
ITERS = CHUNKS_PER_W // NBUF


def _body(x_hbm, tok_hbm, pos_hbm, out_hbm,
          idx_v, r0, r1, r2, r3, pos_v,
          g0, g1, g2, g3, s0, s1, s2, s3):
    rows = [r0, r1, r2, r3]
    gsem = [g0, g1, g2, g3]
    ssem = [s0, s1, s2, s3]

    wid = lax.axis_index("s") * NC + lax.axis_index("c")

    pltpu.sync_copy(x_hbm.at[pl.ds(wid * CHUNKS_PER_W, CHUNKS_PER_W)], idx_v)
    pltpu.sync_copy(pos_hbm.at[pl.ds(wid * POS_PER_W, POS_PER_W)], pos_v)

    def start_gather(c, b):
        pltpu.async_copy(tok_hbm.at[idx_v.at[c]], rows[b], gsem[b])

    def wait_gather(c, b):
        pltpu.make_async_copy(tok_hbm.at[idx_v.at[c]], rows[b], gsem[b]).wait()

    def dst_row(c, k):
        return (c * BB + k) * MAXLEN + wid * POS_PER_W

    def start_store(c, b):
        for k in range(BB):
            pltpu.async_copy(
                rows[b].at[pl.ds(k * POS_PER_W, POS_PER_W)],
                out_hbm.at[pl.ds(dst_row(c, k), POS_PER_W)], ssem[b])

    def wait_store(c, b):
        for k in range(BB):
            pltpu.make_async_copy(
                rows[b].at[pl.ds(k * POS_PER_W, POS_PER_W)],
                out_hbm.at[pl.ds(dst_row(c, k), POS_PER_W)],
                ssem[b]).wait()

    def add_pos(b):
        @plsc.parallel_loop(0, POS_PER_W, unroll=2)
        def _add(t):
            for j in range(EMBED // LANES):
                sl = pl.ds(j * LANES, LANES)
                pv = pos_v[t, sl]
                for k in range(BB):
                    plsc.addupdate(rows[b].at[k * POS_PER_W + t, sl], pv)

    for b in range(AHEAD):
        start_gather(b, b)

    for b in range(NBUF):
        c = b
        wait_gather(c, b)
        add_pos(b)
        start_store(c, b)
        b2 = (b + AHEAD) % NBUF
        if c + AHEAD >= NBUF:
            wait_store(c + AHEAD - NBUF, b2)
        start_gather(c + AHEAD, b2)

    def iter_body(i, carry):
        c0 = i * NBUF
        for b in range(NBUF):
            c = c0 + b
            wait_gather(c, b)
            add_pos(b)
            start_store(c, b)
            b2 = (b + AHEAD) % NBUF

            @pl.when(c + AHEAD < CHUNKS_PER_W)
            def _():
                wait_store(c + AHEAD - NBUF, b2)
                start_gather(c + AHEAD, b2)
        return carry

    lax.fori_loop(1, ITERS, iter_body, 0)

    for b in range(NBUF):
        wait_store(CHUNKS_PER_W - NBUF + b, b)


@jax.jit
def _embed(idx, token_table, pos_table):
    mesh = plsc.VectorSubcoreMesh(core_axis_name="c", subcore_axis_name="s")
    return pl.kernel(
        _body,
        out_type=jax.ShapeDtypeStruct((N_ROWS, EMBED), jnp.float32),
        mesh=mesh,
        scratch_types=[
            pltpu.VMEM((CHUNKS_PER_W, CHUNK), jnp.int32),
        ] + [pltpu.VMEM((CHUNK, EMBED), jnp.float32) for _ in range(NBUF)] + [
            pltpu.VMEM((POS_PER_W, EMBED), jnp.float32),
        ] + [pltpu.SemaphoreType.DMA for _ in range(2 * NBUF)],
    )(idx, token_table, pos_table)


def kernel(x, token_table, pos_table):
    b, m = x.shape
    idx = (x.astype(jnp.int32)
           .reshape(BATCH // BB, BB, NW, POS_PER_W)
           .transpose(2, 0, 1, 3)
           .reshape(NW * CHUNKS_PER_W, CHUNK))
    out = _embed(idx, token_table, pos_table)
    return out.reshape(b, m, EMBED)


# speedup vs baseline: 1.2214x; 1.0282x over previous
import jax
import jax.numpy as jnp
from jax import lax
from jax.experimental import pallas as pl
from jax.experimental.pallas import tpu as pltpu
from jax.experimental.pallas import tpu_sc as plsc

VOCAB = 100000
EMBED = 128
MAXLEN = 512
BATCH = 1024

NC = 2
NS = 16
LANES = 16
NW = NC * NS

N_ROWS = BATCH * MAXLEN
POS_PER_W = MAXLEN // NW
ROWS_PER_W = N_ROWS // NW
BB = 4
CHUNK = BB * POS_PER_W
CHUNKS_PER_W = BATCH // BB
NBUF = 8
AHEAD = 4
ITERS = CHUNKS_PER_W // NBUF


def _body(x_hbm, tok_hbm, pos_hbm, out_hbm,
          idx_v, r0, r1, r2, r3, r4, r5, r6, r7, pos_v,
          g0, g1, g2, g3, g4, g5, g6, g7,
          s0, s1, s2, s3, s4, s5, s6, s7):
    rows = [r0, r1, r2, r3, r4, r5, r6, r7]
    gsem = [g0, g1, g2, g3, g4, g5, g6, g7]
    ssem = [s0, s1, s2, s3, s4, s5, s6, s7]

    wid = lax.axis_index("s") * NC + lax.axis_index("c")

    pltpu.sync_copy(x_hbm.at[pl.ds(wid * CHUNKS_PER_W, CHUNKS_PER_W)], idx_v)
    pltpu.sync_copy(pos_hbm.at[pl.ds(wid * POS_PER_W, POS_PER_W)], pos_v)

    def start_gather(c, b):
        pltpu.async_copy(tok_hbm.at[idx_v.at[c]], rows[b], gsem[b])

    def wait_gather(c, b):
        pltpu.make_async_copy(tok_hbm.at[idx_v.at[c]], rows[b], gsem[b]).wait()

    def dst_row(c, k):
        return (c * BB + k) * MAXLEN + wid * POS_PER_W

    def start_store(c, b):
        for k in range(BB):
            pltpu.async_copy(
                rows[b].at[pl.ds(k * POS_PER_W, POS_PER_W)],
                out_hbm.at[pl.ds(dst_row(c, k), POS_PER_W)], ssem[b])

    def wait_store(c, b):
        for k in range(BB):
            pltpu.make_async_copy(
                rows[b].at[pl.ds(k * POS_PER_W, POS_PER_W)],
                out_hbm.at[pl.ds(dst_row(c, k), POS_PER_W)],
                ssem[b]).wait()

    def add_pos(b):
        @plsc.parallel_loop(0, POS_PER_W, unroll=2)
        def _add(t):
            for j in range(EMBED // LANES):
                sl = pl.ds(j * LANES, LANES)
                pv = pos_v[t, sl]
                for k in range(BB):
                    plsc.addupdate(rows[b].at[k * POS_PER_W + t, sl], pv)

    for b in range(AHEAD):
        start_gather(b, b)

    for b in range(NBUF):
        c = b
        wait_gather(c, b)
        add_pos(b)
        start_store(c, b)
        b2 = (b + AHEAD) % NBUF
        if c + AHEAD >= NBUF:
            wait_store(c + AHEAD - NBUF, b2)
        start_gather(c + AHEAD, b2)

    def iter_body(i, carry):
        c0 = i * NBUF
        for b in range(NBUF):
            c = c0 + b
            wait_gather(c, b)
            add_pos(b)
            start_store(c, b)
            b2 = (b + AHEAD) % NBUF

            @pl.when(c + AHEAD < CHUNKS_PER_W)
            def _():
                wait_store(c + AHEAD - NBUF, b2)
                start_gather(c + AHEAD, b2)
        return carry

    lax.fori_loop(1, ITERS, iter_body, 0)

    for b in range(NBUF):
        wait_store(CHUNKS_PER_W - NBUF + b, b)


@jax.jit
def _embed(idx, token_table, pos_table):
    mesh = plsc.VectorSubcoreMesh(core_axis_name="c", subcore_axis_name="s")
    return pl.kernel(
        _body,
        out_type=jax.ShapeDtypeStruct((N_ROWS, EMBED), jnp.float32),
        mesh=mesh,
        scratch_types=[
            pltpu.VMEM((CHUNKS_PER_W, CHUNK), jnp.int32),
        ] + [pltpu.VMEM((CHUNK, EMBED), jnp.float32) for _ in range(NBUF)] + [
            pltpu.VMEM((POS_PER_W, EMBED), jnp.float32),
        ] + [pltpu.SemaphoreType.DMA for _ in range(2 * NBUF)],
    )(idx, token_table, pos_table)


def kernel(x, token_table, pos_table):
    b, m = x.shape
    idx = (x.astype(jnp.int32)
           .reshape(BATCH // BB, BB, NW, POS_PER_W)
           .transpose(2, 0, 1, 3)
           .reshape(NW * CHUNKS_PER_W, CHUNK))
    out = _embed(idx, token_table, pos_table)
    return out.reshape(b, m, EMBED)
